# baseline (speedup 1.0000x reference)
import functools

import jax
import jax.numpy as jnp
from jax import lax
from jax.experimental import pallas as pl
from jax.experimental.pallas import tpu as pltpu
from jax.experimental.pallas import tpu_sc as plsc

B = 4096
L = 200
D = 64
LP = L + 2
NC, NS = 2, 16
NW = NC * NS
NT = B // NW
DG = D // 8


def _make_gather():
    mesh = plsc.VectorSubcoreMesh(core_axis_name="c", subcore_axis_name="s")

    @functools.partial(
        pl.kernel,
        out_type=jax.ShapeDtypeStruct((LP, DG, NW, 8, 128), jnp.float32),
        mesh=mesh,
        scratch_types=[
            pltpu.VMEM((NT, L), jnp.int32),
            pltpu.VMEM((L, NT), jnp.int32),
            pltpu.VMEM((2, NT, D), jnp.float32),
            pltpu.VMEM((2, DG, 8, 128), jnp.float32),
            pltpu.VMEM((8, 128), jnp.float32),
            pltpu.SemaphoreType.DMA,
            pltpu.SemaphoreType.DMA,
            pltpu.SemaphoreType.DMA,
        ],
        compiler_params=pltpu.CompilerParams(
            use_tc_tiling_on_sc=False, needs_layout_passes=False),
    )
    def gather_kernel(idx_hbm, table_hbm, out_hbm,
                      idx_v, idx_t, emb, blk, zblk, gsem, wsem0, wsem1):
        w = lax.axis_index("s") * NC + lax.axis_index("c")
        wsems = (wsem0, wsem1)
        iot = lax.iota(jnp.int32, 16)

        pltpu.sync_copy(idx_hbm.at[pl.ds(w * NT, NT)], idx_v)

        zf = jnp.zeros((16,), jnp.float32)
        for r in range(8):
            for j in range(8):
                zblk[r, pl.ds(16 * j, 16)] = zf
        for dg in range(DG):
            pltpu.sync_copy(zblk, out_hbm.at[0, dg, w])
            pltpu.sync_copy(zblk, out_hbm.at[L + 1, dg, w])

        def tr_idx(l, carry):
            for j in range(NT // 16):
                rows = iot + (16 * j)
                cols = jnp.full((16,), l, jnp.int32)
                idx_t[l, pl.ds(16 * j, 16)] = plsc.load_gather(
                    idx_v, [rows, cols])
            return carry
        lax.fori_loop(0, L, tr_idx, 0)

        def fire_gather(l, e):
            pltpu.async_copy(table_hbm.at[idx_t.at[l]], emb.at[e], gsem)

        def drain_gather(e):
            pltpu.make_async_copy(
                table_hbm.at[idx_t.at[0]], emb.at[e], gsem).wait()

        def fire_write(l, e):
            for dg in range(DG):
                pltpu.async_copy(blk.at[e, dg], out_hbm.at[l + 1, dg, w],
                                 wsems[e])

        def drain_write(e):
            for dg in range(DG):
                pltpu.make_async_copy(
                    blk.at[e, dg], out_hbm.at[0, dg, 0], wsems[e]).wait()

        def transpose_block(e):
            def tr_dg(dg, carry):
                for dr in range(8):
                    cols = jnp.full((16,), 8 * dg + dr, jnp.int32)
                    for j in range(NT // 16):
                        rows = iot + (16 * j)
                        blk[e, dg, dr, pl.ds(16 * j, 16)] = plsc.load_gather(
                            emb.at[e], [rows, cols])
                return carry
            lax.fori_loop(0, DG, tr_dg, 0)

        fire_gather(0, 0)

        def body(h, carry):
            for b in range(2):
                l = 2 * h + b
                drain_gather(b)
                @pl.when(l <= L - 2)
                def _():
                    fire_gather(l + 1, 1 - b)
                @pl.when(h >= 1)
                def _():
                    drain_write(b)
                transpose_block(b)
                fire_write(l, b)
            return carry

        lax.fori_loop(0, L // 2, body, 0)
        drain_write(0)
        drain_write(1)

    return gather_kernel


_gather = _make_gather()


def kernel(indices, table):
    out5 = _gather(indices, table)
    val_inp = jnp.transpose(out5, (2, 4, 0, 1, 3)).reshape(B, LP, D)
    val_len = jnp.full((B,), LP, dtype=jnp.int32)
    return val_inp, val_len


# speedup vs baseline: 1.3509x; 1.3509x over previous
import functools

import jax
import jax.numpy as jnp
from jax import lax
from jax.experimental import pallas as pl
from jax.experimental.pallas import tpu as pltpu
from jax.experimental.pallas import tpu_sc as plsc

B = 4096
L = 200
D = 64
LP = L + 2
LS = 208
NC, NS = 2, 16
NW = NC * NS
SEQ_PER_W = B // NW
C = 4
G = SEQ_PER_W // C
W0, W1 = 104, 96


def _make_gather():
    mesh = plsc.VectorSubcoreMesh(core_axis_name="c", subcore_axis_name="s")

    @functools.partial(
        pl.kernel,
        out_type=jax.ShapeDtypeStruct((B, LS, D), jnp.float32),
        mesh=mesh,
        scratch_types=[
            pltpu.VMEM((2 * C, W0), jnp.int32),
            pltpu.VMEM((C, LS, D), jnp.float32),
            pltpu.VMEM((C, LS, D), jnp.float32),
            pltpu.SemaphoreType.DMA,
            pltpu.SemaphoreType.DMA,
            pltpu.SemaphoreType.DMA,
        ],
        compiler_params=pltpu.CompilerParams(use_tc_tiling_on_sc=False),
    )
    def gather_kernel(idx_hbm, table_hbm, out_hbm,
                      idx_v, pad0, pad1, gsem, wsem0, wsem1):
        wid = lax.axis_index("s") * NC + lax.axis_index("c")
        pads = (pad0, pad1)
        wsems = (wsem0, wsem1)

        zeros = jnp.zeros((16,), jnp.float32)
        for pv in pads:
            for c in range(C):
                for r in (0, L + 1):
                    for j in range(D // 16):
                        pv[c, r, pl.ds(j * 16, 16)] = zeros

        def step(g, b):
            pv = pads[b]
            seq0 = wid * SEQ_PER_W + g * C
            pltpu.sync_copy(idx_hbm.at[pl.ds(seq0, C), pl.ds(0, W0)],
                            idx_v.at[pl.ds(0, C)])
            pltpu.sync_copy(idx_hbm.at[pl.ds(seq0, C), pl.ds(W0, W1)],
                            idx_v.at[pl.ds(C, C), pl.ds(0, W1)])
            copies = []
            for c in range(C):
                copies.append(pltpu.async_copy(
                    table_hbm.at[idx_v.at[c]],
                    pv.at[c, pl.ds(1, W0)], gsem))
                copies.append(pltpu.async_copy(
                    table_hbm.at[idx_v.at[C + c, pl.ds(0, W1)]],
                    pv.at[c, pl.ds(1 + W0, W1)], gsem))
            for cp in copies:
                cp.wait()
            pltpu.async_copy(pv, out_hbm.at[pl.ds(seq0, C)], wsems[b])

        def drain(b):
            pltpu.make_async_copy(
                pads[b], out_hbm.at[pl.ds(0, C)], wsems[b]).wait()

        def body(h, carry):
            for b in range(2):
                @pl.when(h >= 1)
                def _():
                    drain(b)
                step(2 * h + b, b)
            return carry

        lax.fori_loop(0, G // 2, body, 0)
        drain(0)
        drain(1)

    return gather_kernel


_gather = _make_gather()


def kernel(indices, table):
    x = _gather(indices, table)
    val_inp = (x.reshape(B, LS * D // 128, 128)[:, :LP * D // 128]
               .reshape(B, LP * D // 128, 2, D)
               .reshape(B, LP, D))
    val_len = jnp.full((B,), LP, dtype=jnp.int32)
    return val_inp, val_len


# speedup vs baseline: 1.5327x; 1.1345x over previous
import functools

import jax
import jax.numpy as jnp
from jax import lax
from jax.experimental import pallas as pl
from jax.experimental.pallas import tpu as pltpu
from jax.experimental.pallas import tpu_sc as plsc

B = 4096
L = 200
D = 64
LP = L + 2
NC, NS = 2, 16
NW = NC * NS
SEQ_PER_W = B // NW
C = 4
G = SEQ_PER_W // C
W0, W1 = 104, 96


def _make_gather():
    mesh = plsc.VectorSubcoreMesh(core_axis_name="c", subcore_axis_name="s")

    @functools.partial(
        pl.kernel,
        out_type=jax.ShapeDtypeStruct((B, LP, D), jnp.float32),
        mesh=mesh,
        scratch_types=[
            pltpu.VMEM((2 * C, W0), jnp.int32),
            pltpu.VMEM((C, LP, D), jnp.float32),
            pltpu.VMEM((C, LP, D), jnp.float32),
            pltpu.SemaphoreType.DMA,
            pltpu.SemaphoreType.DMA,
            pltpu.SemaphoreType.DMA,
        ],
        compiler_params=pltpu.CompilerParams(use_tc_tiling_on_sc=False),
    )
    def gather_kernel(idx_hbm, table_hbm, out_hbm,
                      idx_v, pad0, pad1, gsem, wsem0, wsem1):
        wid = lax.axis_index("s") * NC + lax.axis_index("c")
        pads = (pad0, pad1)
        wsems = (wsem0, wsem1)

        zeros = jnp.zeros((16,), jnp.float32)
        for pv in pads:
            for c in range(C):
                for r in (0, L + 1):
                    for j in range(D // 16):
                        pv[c, r, pl.ds(j * 16, 16)] = zeros

        def step(g, b):
            pv = pads[b]
            seq0 = wid * SEQ_PER_W + g * C
            pltpu.sync_copy(idx_hbm.at[pl.ds(seq0, C), pl.ds(0, W0)],
                            idx_v.at[pl.ds(0, C)])
            pltpu.sync_copy(idx_hbm.at[pl.ds(seq0, C), pl.ds(W0, W1)],
                            idx_v.at[pl.ds(C, C), pl.ds(0, W1)])
            copies = []
            for c in range(C):
                copies.append(pltpu.async_copy(
                    table_hbm.at[idx_v.at[c]],
                    pv.at[c, pl.ds(1, W0)], gsem))
                copies.append(pltpu.async_copy(
                    table_hbm.at[idx_v.at[C + c, pl.ds(0, W1)]],
                    pv.at[c, pl.ds(1 + W0, W1)], gsem))
            for cp in copies:
                cp.wait()
            pltpu.async_copy(pv, out_hbm.at[pl.ds(seq0, C)], wsems[b])

        def drain(b):
            pltpu.make_async_copy(
                pads[b], out_hbm.at[pl.ds(0, C)], wsems[b]).wait()

        def body(h, carry):
            for b in range(2):
                @pl.when(h >= 1)
                def _():
                    drain(b)
                step(2 * h + b, b)
            return carry

        lax.fori_loop(0, G // 2, body, 0)
        drain(0)
        drain(1)

    return gather_kernel


_gather = _make_gather()


def kernel(indices, table):
    val_inp = _gather(indices, table)
    val_len = jnp.full((B,), LP, dtype=jnp.int32)
    return val_inp, val_len
